# Initial kernel scaffold; baseline (speedup 1.0000x reference)
import jax
import jax.numpy as jnp
from jax.experimental import pallas as pl


def kernel(node_features, edge_features, global_features, senders, receivers, We1, be1, We2, be2, Wn1, bn1, Wn2, bn2, Wg1, bg1, Wg2, bg2):
    raise NotImplementedError("write your pallas kernel here")


# speedup vs baseline: 3.6101x; 3.6101x over previous
import functools

import jax
import jax.numpy as jnp
from jax import lax
from jax.experimental import pallas as pl
from jax.experimental.pallas import tpu as pltpu
from jax.experimental.pallas import tpu_sc as plsc

N = 10000
E = 320000
DN = 128
DE = 16
DG = 128
H = 128

NC = 2
NS = 16
NW = NC * NS
EPW = E // NW
CH = 80
NCH = EPW // CH

NP = 10240
RPT = NP // NS

BIG = 2000
NBIG = E // BIG
BPW = NBIG // NS
NIN = BIG // CH

TN = 1000
TE = 2000


def _proj_body(nf, we_s, we_r, g, we_g, be1, wn_g, bn1,
               p_out, q_out, ce_out, cn_out):
    i = pl.program_id(0)
    x = nf[...]
    p_out[...] = jnp.dot(x, we_s[...], preferred_element_type=jnp.float32)
    q_out[...] = jnp.dot(x, we_r[...], preferred_element_type=jnp.float32)

    @pl.when(i == 0)
    def _():
        gv = g[...]
        ce_out[...] = jnp.dot(gv, we_g[...],
                              preferred_element_type=jnp.float32) + be1[...]
        cn_out[...] = jnp.dot(gv, wn_g[...],
                              preferred_element_type=jnp.float32) + bn1[...]


def _projections(nf, we_s, we_r, g, we_g, be1, wn_g, bn1):
    grid = (N // TN,)
    full = lambda shape: pl.BlockSpec(shape, lambda i: (0, 0))
    return pl.pallas_call(
        _proj_body,
        grid=grid,
        in_specs=[
            pl.BlockSpec((TN, DN), lambda i: (i, 0)),
            full((DN, H)), full((DN, H)), full((1, DG)), full((DG, H)),
            full((1, H)), full((DG, H)), full((1, H)),
        ],
        out_specs=[
            pl.BlockSpec((TN, H), lambda i: (i, 0)),
            pl.BlockSpec((TN, H), lambda i: (i, 0)),
            full((1, H)), full((1, H)),
        ],
        out_shape=[
            jax.ShapeDtypeStruct((N, H), jnp.float32),
            jax.ShapeDtypeStruct((N, H), jnp.float32),
            jax.ShapeDtypeStruct((1, H), jnp.float32),
            jax.ShapeDtypeStruct((1, H), jnp.float32),
        ],
    )(nf, we_s, we_r, g, we_g, be1, wn_g, bn1)


def _sc_gather_body(p_hbm, q_hbm, sidx_hbm, ridx_hbm, gp_out, gq_out,
                    sidx_v, ridx_v, rp, rq, sem):
    cid = lax.axis_index("c")
    sid = lax.axis_index("s")
    wid = sid * NC + cid
    pltpu.sync_copy(sidx_hbm.at[wid], sidx_v)
    pltpu.sync_copy(ridx_hbm.at[wid], ridx_v)

    def body(j, carry):
        pltpu.async_copy(p_hbm.at[sidx_v.at[j]], rp, sem).wait()
        pltpu.sync_copy(rp, gp_out.at[wid * NCH + j])
        pltpu.async_copy(q_hbm.at[ridx_v.at[j]], rq, sem).wait()
        pltpu.sync_copy(rq, gq_out.at[wid * NCH + j])
        return carry

    lax.fori_loop(0, NCH, body, 0)


def _sc_gather(p, q, sidx3, ridx3):
    mesh = plsc.VectorSubcoreMesh(core_axis_name="c", subcore_axis_name="s")
    out = pl.kernel(
        _sc_gather_body,
        out_type=[
            jax.ShapeDtypeStruct((E // CH, CH, DN), jnp.float32),
            jax.ShapeDtypeStruct((E // CH, CH, DN), jnp.float32),
        ],
        mesh=mesh,
        scratch_types=[
            pltpu.VMEM((NCH, CH), jnp.int32),
            pltpu.VMEM((NCH, CH), jnp.int32),
            pltpu.VMEM((CH, DN), jnp.float32),
            pltpu.VMEM((CH, DN), jnp.float32),
            pltpu.SemaphoreType.DMA,
        ],
        compiler_params=pltpu.CompilerParams(use_tc_tiling_on_sc=False),
    )(p, q, sidx3, ridx3)
    return out


def _edge_body(ef, gp, gq, we_e, ce, we2, be2, ne_out, e2g_out, acc):
    i = pl.program_id(0)
    pre = jnp.dot(ef[...], we_e[...], preferred_element_type=jnp.float32)
    pre = pre + gp[...] + gq[...] + ce[...]
    he = jnp.maximum(pre, 0.0)
    ne = jnp.dot(he, we2[...], preferred_element_type=jnp.float32) + be2[...]
    ne_out[...] = ne
    part = jnp.sum(ne, axis=0, keepdims=True)

    @pl.when(i == 0)
    def _():
        acc[...] = part

    @pl.when(i > 0)
    def _():
        acc[...] = acc[...] + part

    @pl.when(i == pl.num_programs(0) - 1)
    def _():
        e2g_out[...] = acc[...]


def _edge_mlp(ef, gp, gq, we_e, ce, we2, be2):
    grid = (E // TE,)
    full = lambda shape: pl.BlockSpec(shape, lambda i: (0, 0))
    return pl.pallas_call(
        _edge_body,
        grid=grid,
        in_specs=[
            pl.BlockSpec((TE, DE), lambda i: (i, 0)),
            pl.BlockSpec((TE, DN), lambda i: (i, 0)),
            pl.BlockSpec((TE, DN), lambda i: (i, 0)),
            full((DE, H)), full((1, H)), full((H, DE)), full((1, DE)),
        ],
        out_specs=[
            pl.BlockSpec((TE, DE), lambda i: (i, 0)),
            full((1, DE)),
        ],
        out_shape=[
            jax.ShapeDtypeStruct((E, DE), jnp.float32),
            jax.ShapeDtypeStruct((1, DE), jnp.float32),
        ],
        scratch_shapes=[pltpu.VMEM((1, DE), jnp.float32)],
    )(ef, gp, gq, we_e, ce, we2, be2)


def _sc_scatter_body(ne_hbm, idx_hbm, zer_hbm, agg_out, table, ne_buf, idx_v):
    cid = lax.axis_index("c")
    sid = lax.axis_index("s")
    pltpu.sync_copy(zer_hbm.at[sid], table.at[pl.ds(sid * RPT, RPT)])
    plsc.subcore_barrier()

    def big(t, carry):
        b = sid * BPW + t
        pltpu.sync_copy(ne_hbm.at[b], ne_buf)
        pltpu.sync_copy(idx_hbm.at[cid, b], idx_v)

        def inner(j, c2):
            pltpu.sync_copy(ne_buf.at[pl.ds(j * CH, CH)],
                            table.at[idx_v.at[j]], add=True)
            return c2

        lax.fori_loop(0, NIN, inner, 0)
        return carry

    lax.fori_loop(0, BPW, big, 0)
    plsc.subcore_barrier()
    pltpu.sync_copy(table.at[pl.ds(sid * RPT, RPT)], agg_out.at[cid, sid])


def _sc_scatter(ne3, idx4, zer3):
    mesh = plsc.VectorSubcoreMesh(core_axis_name="c", subcore_axis_name="s")
    return pl.kernel(
        _sc_scatter_body,
        out_type=jax.ShapeDtypeStruct((2, NS, RPT, DE), jnp.float32),
        mesh=mesh,
        scratch_types=[
            pltpu.VMEM_SHARED((NP, DE), jnp.float32),
            pltpu.VMEM((BIG, DE), jnp.float32),
            pltpu.VMEM((NIN, CH), jnp.int32),
        ],
        compiler_params=pltpu.CompilerParams(use_tc_tiling_on_sc=False),
    )(ne3, idx4, zer3)


def _node_body(nf, ags, agr, cn, wn_n, wn_s, wn_r, wn2, bn2,
               g, e2g, wg_g, wg_n, wg_e, bg1, wg2, bg2,
               nn_out, ng_out, nacc):
    i = pl.program_id(0)
    pre = (jnp.dot(nf[...], wn_n[...], preferred_element_type=jnp.float32)
           + jnp.dot(ags[...], wn_s[...], preferred_element_type=jnp.float32)
           + jnp.dot(agr[...], wn_r[...], preferred_element_type=jnp.float32)
           + cn[...])
    hn = jnp.maximum(pre, 0.0)
    nn = jnp.dot(hn, wn2[...], preferred_element_type=jnp.float32) + bn2[...]
    nn_out[...] = nn
    part = jnp.sum(nn, axis=0, keepdims=True)

    @pl.when(i == 0)
    def _():
        nacc[...] = part

    @pl.when(i > 0)
    def _():
        nacc[...] = nacc[...] + part

    @pl.when(i == pl.num_programs(0) - 1)
    def _():
        gpre = (jnp.dot(g[...], wg_g[...], preferred_element_type=jnp.float32)
                + jnp.dot(nacc[...], wg_n[...],
                          preferred_element_type=jnp.float32)
                + jnp.dot(e2g[...], wg_e[...],
                          preferred_element_type=jnp.float32)
                + bg1[...])
        hg = jnp.maximum(gpre, 0.0)
        ng_out[...] = jnp.dot(hg, wg2[...],
                              preferred_element_type=jnp.float32) + bg2[...]


def _node_mlp(nf, ags, agr, cn, wn_n, wn_s, wn_r, wn2, bn2,
              g, e2g, wg_g, wg_n, wg_e, bg1, wg2, bg2):
    grid = (N // TN,)
    full = lambda shape: pl.BlockSpec(shape, lambda i: (0, 0))
    return pl.pallas_call(
        _node_body,
        grid=grid,
        in_specs=[
            pl.BlockSpec((TN, DN), lambda i: (i, 0)),
            pl.BlockSpec((TN, DE), lambda i: (i, 0)),
            pl.BlockSpec((TN, DE), lambda i: (i, 0)),
            full((1, H)), full((DN, H)), full((DE, H)), full((DE, H)),
            full((H, DN)), full((1, DN)),
            full((1, DG)), full((1, DE)),
            full((DG, H)), full((DN, H)), full((DE, H)), full((1, H)),
            full((H, DG)), full((1, DG)),
        ],
        out_specs=[
            pl.BlockSpec((TN, DN), lambda i: (i, 0)),
            full((1, DG)),
        ],
        out_shape=[
            jax.ShapeDtypeStruct((N, DN), jnp.float32),
            jax.ShapeDtypeStruct((1, DG), jnp.float32),
        ],
        scratch_shapes=[pltpu.VMEM((1, DN), jnp.float32)],
    )(nf, ags, agr, cn, wn_n, wn_s, wn_r, wn2, bn2,
      g, e2g, wg_g, wg_n, wg_e, bg1, wg2, bg2)


def kernel(node_features, edge_features, global_features, senders, receivers,
           We1, be1, We2, be2, Wn1, bn1, Wn2, bn2, Wg1, bg1, Wg2, bg2):
    we_e = We1[0:DE]
    we_s = We1[DE:DE + DN]
    we_r = We1[DE + DN:DE + 2 * DN]
    we_g = We1[DE + 2 * DN:]
    wn_n = Wn1[0:DN]
    wn_s = Wn1[DN:DN + DE]
    wn_r = Wn1[DN + DE:DN + 2 * DE]
    wn_g = Wn1[DN + 2 * DE:]
    wg_g = Wg1[0:DG]
    wg_n = Wg1[DG:DG + DN]
    wg_e = Wg1[DG + DN:]

    p, q, ce, cn = _projections(
        node_features, we_s, we_r, global_features, we_g,
        be1.reshape(1, H), wn_g, bn1.reshape(1, H))

    sidx3 = senders.reshape(NW, NCH, CH)
    ridx3 = receivers.reshape(NW, NCH, CH)
    gp3, gq3 = _sc_gather(p, q, sidx3, ridx3)

    new_edges, e2g = _edge_mlp(
        edge_features, gp3.reshape(E, DN), gq3.reshape(E, DN),
        we_e, ce, We2, be2.reshape(1, DE))

    ne3 = new_edges.reshape(NBIG, BIG, DE)
    idx4 = jnp.stack([senders.reshape(NBIG, NIN, CH),
                      receivers.reshape(NBIG, NIN, CH)])
    zer3 = jnp.zeros((NS, RPT, DE), jnp.float32)
    agg4 = _sc_scatter(ne3, idx4, zer3)
    agg = agg4.reshape(2, NP, DE)
    ags = agg[0, :N]
    agr = agg[1, :N]

    new_nodes, new_global = _node_mlp(
        node_features, ags, agr, cn, wn_n, wn_s, wn_r, Wn2,
        bn2.reshape(1, DN), global_features, e2g,
        wg_g, wg_n, wg_e, bg1.reshape(1, H), Wg2, bg2.reshape(1, DG))

    return (new_nodes, new_edges, new_global)
